# baseline (speedup 1.0000x reference)
import functools

import jax
import jax.numpy as jnp
from jax import lax
from jax.experimental import pallas as pl
from jax.experimental.pallas import tpu as pltpu
from jax.experimental.pallas import tpu_sc as plsc

_BATCH = 16384
_D = 32
_NW = 32
_BPW = _BATCH // _NW
_CHUNK = 128
_NCHUNK = _BPW // _CHUNK

_N = 1000000
_NBLK = (_N + 127) // 128
_BLK_PER_W = (_NBLK + 31) // 32
_TAIL = _N - (_NBLK - 1) * 128


def _tr_kernel(src_hbm, dst_hbm, in0, in1, patch0, patch1, sem_in, sem_out):
    wid = lax.axis_index("s") * 2 + lax.axis_index("c")
    lo = wid * _BLK_PER_W
    hi = jnp.minimum(lo + _BLK_PER_W, _NBLK)

    ins = [in0, in1]
    patches = [patch0, patch1]
    iota = lax.iota(jnp.int32, 16)

    def fire_in(c, buf):
        col0 = pl.multiple_of(c * 128, 128)
        pltpu.make_async_copy(
            src_hbm.at[:, pl.ds(col0, 128)], buf, sem_in).start()

    def wait_in(buf):
        pltpu.make_async_copy(
            src_hbm.at[:, pl.ds(0, 128)], buf, sem_in).wait()

    def transpose(buf, patch):
        def step(c8, _):
            for u in range(8):
                rv = c8 * 8 + u + iota
                rv = jnp.where(rv >= 128, rv - 128, rv)
                v0 = plsc.load_gather(buf, [iota, rv])
                v1 = plsc.load_gather(buf, [iota + 16, rv])
                plsc.store_scatter(patch, [rv, iota], v0)
                plsc.store_scatter(patch, [rv, iota + 16], v1)
            return 0
        lax.fori_loop(0, 16, step, 0)

    def fire_out(c, patch, is_tail):
        row0 = pl.multiple_of(c * 128, 8)

        @pl.when(jnp.logical_not(is_tail))
        def _():
            pltpu.make_async_copy(
                patch, dst_hbm.at[pl.ds(row0, 128), :], sem_out).start()

        @pl.when(is_tail)
        def _():
            pltpu.make_async_copy(
                patch.at[pl.ds(0, _TAIL), :],
                dst_hbm.at[pl.ds(row0, _TAIL), :], sem_out).start()

    def drain_out(nrows):
        pltpu.make_async_copy(
            patch0.at[pl.ds(0, nrows), :],
            dst_hbm.at[pl.ds(0, nrows), :], sem_out).wait()

    @pl.when(lo < hi)
    def _():
        fire_in(lo, ins[0])

    def body(k, _):
        for bufi in range(2):
            c = lo + 2 * k + bufi

            @pl.when(c < hi)
            def _():
                is_tail = c == _NBLK - 1
                wait_in(ins[bufi])

                @pl.when(c + 1 < hi)
                def _():
                    fire_in(c + 1, ins[1 - bufi])

                @pl.when(c - 2 >= lo)
                def _():
                    drain_out(128)

                transpose(ins[bufi], patches[bufi])
                fire_out(c, patches[bufi], is_tail)
        return 0

    lax.fori_loop(0, (_BLK_PER_W + 1) // 2, body, 0)

    nb = hi - lo

    @pl.when((nb >= 2) & (hi == _NBLK))
    def _():
        drain_out(128)
        drain_out(_TAIL)

    @pl.when((nb >= 2) & (hi < _NBLK))
    def _():
        drain_out(128)
        drain_out(128)

    @pl.when((nb == 1) & (hi == _NBLK))
    def _():
        drain_out(_TAIL)

    @pl.when((nb == 1) & (hi < _NBLK))
    def _():
        drain_out(128)


def _sc_transpose_one(table_t):
    mesh = plsc.VectorSubcoreMesh(core_axis_name="c", subcore_axis_name="s")
    run = pl.kernel(
        _tr_kernel,
        out_type=jax.ShapeDtypeStruct((_N, _D), jnp.float32),
        mesh=mesh,
        scratch_types=[
            pltpu.VMEM((_D, 128), jnp.float32),
            pltpu.VMEM((_D, 128), jnp.float32),
            pltpu.VMEM((128, _D), jnp.float32),
            pltpu.VMEM((128, _D), jnp.float32),
            pltpu.SemaphoreType.DMA,
            pltpu.SemaphoreType.DMA,
        ],
        compiler_params=pltpu.CompilerParams(
            use_tc_tiling_on_sc=True, needs_layout_passes=False,
            disable_bounds_checks=True),
    )
    return run(table_t)


def _sc_kernel(sid_hbm, pid_hbm, sw_hbm, pw_hbm, out_hbm,
               sidx_v, pidx_v, srows_v, prows_v, out_v, sem):
    wid = lax.axis_index("s") * 2 + lax.axis_index("c")

    pltpu.sync_copy(sid_hbm.at[wid], sidx_v)
    pltpu.sync_copy(pid_hbm.at[wid], pidx_v)

    copies = []
    for j in range(_NCHUNK):
        sl = pl.ds(j * _CHUNK, _CHUNK)
        copies.append(pltpu.make_async_copy(sw_hbm.at[sidx_v.at[j]],
                                            srows_v.at[sl], sem))
        copies.append(pltpu.make_async_copy(pw_hbm.at[pidx_v.at[j]],
                                            prows_v.at[sl], sem))
    for c in copies:
        c.start()
    for c in copies:
        c.wait()

    iota = lax.iota(jnp.int32, 16)

    def body(g, _):
        rowv = g * 16 + iota

        acc = jnp.zeros((16,), jnp.float32)
        for d in range(_D):
            colv = iota + d
            colv = jnp.where(colv >= _D, colv - _D, colv)
            vs = plsc.load_gather(srows_v, [rowv, colv])
            vp = plsc.load_gather(prows_v, [rowv, colv])
            acc = acc + vs * vp
        out_v[pl.ds(g * 16, 16)] = acc
        return 0

    lax.fori_loop(0, _BPW // 16, body, 0)

    pltpu.sync_copy(out_v, out_hbm.at[wid])


def _sc_gather_dot(sid3, pid3, sw_lin, pw_lin):
    mesh = plsc.VectorSubcoreMesh(core_axis_name="c", subcore_axis_name="s")
    run = pl.kernel(
        _sc_kernel,
        out_type=jax.ShapeDtypeStruct((_NW, _BPW), jnp.float32),
        mesh=mesh,
        scratch_types=[
            pltpu.VMEM((_NCHUNK, _CHUNK), jnp.int32),
            pltpu.VMEM((_NCHUNK, _CHUNK), jnp.int32),
            pltpu.VMEM((_BPW, _D), jnp.float32),
            pltpu.VMEM((_BPW, _D), jnp.float32),
            pltpu.VMEM((_BPW,), jnp.float32),
            pltpu.SemaphoreType.DMA,
        ],
        compiler_params=pltpu.CompilerParams(
            use_tc_tiling_on_sc=False, needs_layout_passes=False),
    )
    return run(sid3, pid3, sw_lin, pw_lin)


def kernel(sid, pid, scientist_weight, paper_weight):
    sid3 = sid.astype(jnp.int32).reshape(_NW, _NCHUNK, _CHUNK)
    pid3 = pid.astype(jnp.int32).reshape(_NW, _NCHUNK, _CHUNK)

    sw_lin = _sc_transpose_one(scientist_weight.T)
    pw_lin = _sc_transpose_one(paper_weight.T)

    out = _sc_gather_dot(sid3, pid3, sw_lin, pw_lin)
    return out.reshape(_BATCH)


# speedup vs baseline: 3.4126x; 3.4126x over previous
import functools

import jax
import jax.numpy as jnp
from jax import lax
from jax.experimental import pallas as pl
from jax.experimental.pallas import tpu as pltpu
from jax.experimental.pallas import tpu_sc as plsc

_BATCH = 16384
_D = 32
_NW = 32
_BPW = _BATCH // _NW
_CHUNK = 128
_NCHUNK = _BPW // _CHUNK

_N = 1000000
_SB = 512
_NSB = (_N + _SB - 1) // _SB
_SB_PER_W = (_NSB + 31) // 32
_TAIL = _N - (_NSB - 1) * _SB


def _tr_kernel(src_hbm, dst_hbm, in0, in1, patch0, patch1, sem_in, sem_out):
    wid = lax.axis_index("s") * 2 + lax.axis_index("c")
    lo = wid * _SB_PER_W
    hi = jnp.minimum(lo + _SB_PER_W, _NSB)

    ins = [in0, in1]
    patches = [patch0, patch1]
    iota = lax.iota(jnp.int32, 16)

    def fire_in(c, buf):
        col0 = pl.multiple_of(c * _SB, 128)
        is_tail = c == _NSB - 1

        @pl.when(jnp.logical_not(is_tail))
        def _():
            pltpu.make_async_copy(
                src_hbm.at[:, pl.ds(col0, _SB)], buf, sem_in).start()

        @pl.when(is_tail)
        def _():
            pltpu.make_async_copy(
                src_hbm.at[:, pl.ds(col0, 128)],
                buf.at[:, pl.ds(0, 128)], sem_in).start()

    def wait_in(c, buf):
        is_tail = c == _NSB - 1

        @pl.when(jnp.logical_not(is_tail))
        def _():
            pltpu.make_async_copy(
                src_hbm.at[:, pl.ds(0, _SB)], buf, sem_in).wait()

        @pl.when(is_tail)
        def _():
            pltpu.make_async_copy(
                src_hbm.at[:, pl.ds(0, 128)],
                buf.at[:, pl.ds(0, 128)], sem_in).wait()

    def transpose(buf, patch):
        def step(c8, _):
            for u in range(8):
                rv = c8 * 8 + u + iota
                rv = jnp.where(rv >= _SB, rv - _SB, rv)
                v0 = plsc.load_gather(buf, [iota, rv])
                v1 = plsc.load_gather(buf, [iota + 16, rv])
                plsc.store_scatter(patch, [rv * _D + iota], v0)
                plsc.store_scatter(patch, [rv * _D + 16 + iota], v1)
            return 0
        lax.fori_loop(0, _SB // 8, step, 0)

    def fire_out(c, patch):
        off = pl.multiple_of(c * _SB * _D, 8)
        is_tail = c == _NSB - 1

        @pl.when(jnp.logical_not(is_tail))
        def _():
            pltpu.make_async_copy(
                patch, dst_hbm.at[pl.ds(off, _SB * _D)], sem_out).start()

        @pl.when(is_tail)
        def _():
            pltpu.make_async_copy(
                patch.at[pl.ds(0, _TAIL * _D)],
                dst_hbm.at[pl.ds(off, _TAIL * _D)], sem_out).start()

    def drain_out(nrows):
        pltpu.make_async_copy(
            patch0.at[pl.ds(0, nrows * _D)],
            dst_hbm.at[pl.ds(0, nrows * _D)], sem_out).wait()

    @pl.when(lo < hi)
    def _():
        fire_in(lo, ins[0])

    def body(k, _):
        for bufi in range(2):
            c = lo + 2 * k + bufi

            @pl.when(c < hi)
            def _():
                wait_in(c, ins[bufi])

                @pl.when(c + 1 < hi)
                def _():
                    fire_in(c + 1, ins[1 - bufi])

                @pl.when(c - 2 >= lo)
                def _():
                    drain_out(_SB)

                transpose(ins[bufi], patches[bufi])
                fire_out(c, patches[bufi])
        return 0

    lax.fori_loop(0, (_SB_PER_W + 1) // 2, body, 0)

    nb = hi - lo

    @pl.when((nb >= 2) & (hi == _NSB))
    def _():
        drain_out(_SB)
        drain_out(_TAIL)

    @pl.when((nb >= 2) & (hi < _NSB))
    def _():
        drain_out(_SB)
        drain_out(_SB)

    @pl.when((nb == 1) & (hi == _NSB))
    def _():
        drain_out(_TAIL)

    @pl.when((nb == 1) & (hi < _NSB))
    def _():
        drain_out(_SB)


def _sc_transpose_one(table_t):
    mesh = plsc.VectorSubcoreMesh(core_axis_name="c", subcore_axis_name="s")
    run = pl.kernel(
        _tr_kernel,
        out_type=jax.ShapeDtypeStruct((_N * _D,), jnp.float32),
        mesh=mesh,
        scratch_types=[
            pltpu.VMEM((_D, _SB), jnp.float32),
            pltpu.VMEM((_D, _SB), jnp.float32),
            pltpu.VMEM((_SB * _D,), jnp.float32),
            pltpu.VMEM((_SB * _D,), jnp.float32),
            pltpu.SemaphoreType.DMA,
            pltpu.SemaphoreType.DMA,
        ],
        compiler_params=pltpu.CompilerParams(
            use_tc_tiling_on_sc=True, needs_layout_passes=False,
            disable_bounds_checks=True),
    )
    return run(table_t)


def _sc_kernel(sid_hbm, pid_hbm, sw_hbm, pw_hbm, out_hbm,
               sidx_v, pidx_v, srows_v, prows_v, out_v, sem):
    wid = lax.axis_index("s") * 2 + lax.axis_index("c")

    pltpu.sync_copy(sid_hbm.at[wid], sidx_v)
    pltpu.sync_copy(pid_hbm.at[wid], pidx_v)

    copies = []
    for j in range(_NCHUNK):
        sl = pl.ds(j * _CHUNK, _CHUNK)
        copies.append(pltpu.make_async_copy(sw_hbm.at[sidx_v.at[j]],
                                            srows_v.at[sl], sem))
        copies.append(pltpu.make_async_copy(pw_hbm.at[pidx_v.at[j]],
                                            prows_v.at[sl], sem))
    for c in copies:
        c.start()
    for c in copies:
        c.wait()

    iota = lax.iota(jnp.int32, 16)

    def body(g, _):
        rowv = g * 16 + iota

        acc = jnp.zeros((16,), jnp.float32)
        for d in range(_D):
            colv = iota + d
            colv = jnp.where(colv >= _D, colv - _D, colv)
            vs = plsc.load_gather(srows_v, [rowv, colv])
            vp = plsc.load_gather(prows_v, [rowv, colv])
            acc = acc + vs * vp
        out_v[pl.ds(g * 16, 16)] = acc
        return 0

    lax.fori_loop(0, _BPW // 16, body, 0)

    pltpu.sync_copy(out_v, out_hbm.at[wid])


def _sc_gather_dot(sid3, pid3, sw_lin, pw_lin):
    mesh = plsc.VectorSubcoreMesh(core_axis_name="c", subcore_axis_name="s")
    run = pl.kernel(
        _sc_kernel,
        out_type=jax.ShapeDtypeStruct((_NW, _BPW), jnp.float32),
        mesh=mesh,
        scratch_types=[
            pltpu.VMEM((_NCHUNK, _CHUNK), jnp.int32),
            pltpu.VMEM((_NCHUNK, _CHUNK), jnp.int32),
            pltpu.VMEM((_BPW, _D), jnp.float32),
            pltpu.VMEM((_BPW, _D), jnp.float32),
            pltpu.VMEM((_BPW,), jnp.float32),
            pltpu.SemaphoreType.DMA,
        ],
        compiler_params=pltpu.CompilerParams(
            use_tc_tiling_on_sc=False, needs_layout_passes=False),
    )
    return run(sid3, pid3, sw_lin, pw_lin)


def kernel(sid, pid, scientist_weight, paper_weight):
    sid3 = sid.astype(jnp.int32).reshape(_NW, _NCHUNK, _CHUNK)
    pid3 = pid.astype(jnp.int32).reshape(_NW, _NCHUNK, _CHUNK)

    sw_lin = _sc_transpose_one(scientist_weight.T).reshape(_N, _D)
    pw_lin = _sc_transpose_one(paper_weight.T).reshape(_N, _D)

    out = _sc_gather_dot(sid3, pid3, sw_lin, pw_lin)
    return out.reshape(_BATCH)


# speedup vs baseline: 3.4359x; 1.0068x over previous
import functools

import jax
import jax.numpy as jnp
from jax import lax
from jax.experimental import pallas as pl
from jax.experimental.pallas import tpu as pltpu
from jax.experimental.pallas import tpu_sc as plsc

_BATCH = 16384
_D = 32
_NW = 32
_BPW = _BATCH // _NW
_CHUNK = 128
_NCHUNK = _BPW // _CHUNK

_N = 1000000
_SB = 768
_NSB = (_N + _SB - 1) // _SB
_SB_PER_W = (_NSB + 31) // 32
_TAIL = _N - (_NSB - 1) * _SB


def _tr_kernel(src_hbm, dst_hbm, in0, in1, patch0, patch1, sem_in, sem_out):
    wid = lax.axis_index("s") * 2 + lax.axis_index("c")
    lo = wid * _SB_PER_W
    hi = jnp.minimum(lo + _SB_PER_W, _NSB)

    ins = [in0, in1]
    patches = [patch0, patch1]
    iota = lax.iota(jnp.int32, 16)

    def fire_in(c, buf):
        col0 = pl.multiple_of(c * _SB, 128)
        is_tail = c == _NSB - 1

        @pl.when(jnp.logical_not(is_tail))
        def _():
            pltpu.make_async_copy(
                src_hbm.at[:, pl.ds(col0, _SB)], buf, sem_in).start()

        @pl.when(is_tail)
        def _():
            pltpu.make_async_copy(
                src_hbm.at[:, pl.ds(col0, 128)],
                buf.at[:, pl.ds(0, 128)], sem_in).start()

    def wait_in(c, buf):
        is_tail = c == _NSB - 1

        @pl.when(jnp.logical_not(is_tail))
        def _():
            pltpu.make_async_copy(
                src_hbm.at[:, pl.ds(0, _SB)], buf, sem_in).wait()

        @pl.when(is_tail)
        def _():
            pltpu.make_async_copy(
                src_hbm.at[:, pl.ds(0, 128)],
                buf.at[:, pl.ds(0, 128)], sem_in).wait()

    def transpose(buf, patch):
        def step(c8, _):
            for u in range(8):
                rv = c8 * 8 + u + iota
                rv = jnp.where(rv >= _SB, rv - _SB, rv)
                v0 = plsc.load_gather(buf, [iota, rv])
                v1 = plsc.load_gather(buf, [iota + 16, rv])
                plsc.store_scatter(patch, [rv * _D + iota], v0)
                plsc.store_scatter(patch, [rv * _D + 16 + iota], v1)
            return 0
        lax.fori_loop(0, _SB // 8, step, 0)

    def fire_out(c, patch):
        off = pl.multiple_of(c * _SB * _D, 8)
        is_tail = c == _NSB - 1

        @pl.when(jnp.logical_not(is_tail))
        def _():
            pltpu.make_async_copy(
                patch, dst_hbm.at[pl.ds(off, _SB * _D)], sem_out).start()

        @pl.when(is_tail)
        def _():
            pltpu.make_async_copy(
                patch.at[pl.ds(0, _TAIL * _D)],
                dst_hbm.at[pl.ds(off, _TAIL * _D)], sem_out).start()

    def drain_out(nrows):
        pltpu.make_async_copy(
            patch0.at[pl.ds(0, nrows * _D)],
            dst_hbm.at[pl.ds(0, nrows * _D)], sem_out).wait()

    @pl.when(lo < hi)
    def _():
        fire_in(lo, ins[0])

    def body(k, _):
        for bufi in range(2):
            c = lo + 2 * k + bufi

            @pl.when(c < hi)
            def _():
                wait_in(c, ins[bufi])

                @pl.when(c + 1 < hi)
                def _():
                    fire_in(c + 1, ins[1 - bufi])

                @pl.when(c - 2 >= lo)
                def _():
                    drain_out(_SB)

                transpose(ins[bufi], patches[bufi])
                fire_out(c, patches[bufi])
        return 0

    lax.fori_loop(0, (_SB_PER_W + 1) // 2, body, 0)

    nb = hi - lo

    @pl.when((nb >= 2) & (hi == _NSB))
    def _():
        drain_out(_SB)
        drain_out(_TAIL)

    @pl.when((nb >= 2) & (hi < _NSB))
    def _():
        drain_out(_SB)
        drain_out(_SB)

    @pl.when((nb == 1) & (hi == _NSB))
    def _():
        drain_out(_TAIL)

    @pl.when((nb == 1) & (hi < _NSB))
    def _():
        drain_out(_SB)


def _sc_transpose_one(table_t):
    mesh = plsc.VectorSubcoreMesh(core_axis_name="c", subcore_axis_name="s")
    run = pl.kernel(
        _tr_kernel,
        out_type=jax.ShapeDtypeStruct((_N * _D,), jnp.float32),
        mesh=mesh,
        scratch_types=[
            pltpu.VMEM((_D, _SB), jnp.float32),
            pltpu.VMEM((_D, _SB), jnp.float32),
            pltpu.VMEM((_SB * _D,), jnp.float32),
            pltpu.VMEM((_SB * _D,), jnp.float32),
            pltpu.SemaphoreType.DMA,
            pltpu.SemaphoreType.DMA,
        ],
        compiler_params=pltpu.CompilerParams(
            use_tc_tiling_on_sc=True, needs_layout_passes=False,
            disable_bounds_checks=True),
    )
    return run(table_t)


def _sc_kernel(sid_hbm, pid_hbm, sw_hbm, pw_hbm, out_hbm,
               sidx_v, pidx_v, srows_v, prows_v, out_v, sem):
    wid = lax.axis_index("s") * 2 + lax.axis_index("c")

    pltpu.sync_copy(sid_hbm.at[wid], sidx_v)
    pltpu.sync_copy(pid_hbm.at[wid], pidx_v)

    copies = []
    for j in range(_NCHUNK):
        sl = pl.ds(j * _CHUNK, _CHUNK)
        copies.append(pltpu.make_async_copy(sw_hbm.at[sidx_v.at[j]],
                                            srows_v.at[sl], sem))
        copies.append(pltpu.make_async_copy(pw_hbm.at[pidx_v.at[j]],
                                            prows_v.at[sl], sem))
    for c in copies:
        c.start()
    for c in copies:
        c.wait()

    iota = lax.iota(jnp.int32, 16)

    def body(g, _):
        rowv = g * 16 + iota

        acc = jnp.zeros((16,), jnp.float32)
        for d in range(_D):
            colv = iota + d
            colv = jnp.where(colv >= _D, colv - _D, colv)
            vs = plsc.load_gather(srows_v, [rowv, colv])
            vp = plsc.load_gather(prows_v, [rowv, colv])
            acc = acc + vs * vp
        out_v[pl.ds(g * 16, 16)] = acc
        return 0

    lax.fori_loop(0, _BPW // 16, body, 0)

    pltpu.sync_copy(out_v, out_hbm.at[wid])


def _sc_gather_dot(sid3, pid3, sw_lin, pw_lin):
    mesh = plsc.VectorSubcoreMesh(core_axis_name="c", subcore_axis_name="s")
    run = pl.kernel(
        _sc_kernel,
        out_type=jax.ShapeDtypeStruct((_NW, _BPW), jnp.float32),
        mesh=mesh,
        scratch_types=[
            pltpu.VMEM((_NCHUNK, _CHUNK), jnp.int32),
            pltpu.VMEM((_NCHUNK, _CHUNK), jnp.int32),
            pltpu.VMEM((_BPW, _D), jnp.float32),
            pltpu.VMEM((_BPW, _D), jnp.float32),
            pltpu.VMEM((_BPW,), jnp.float32),
            pltpu.SemaphoreType.DMA,
        ],
        compiler_params=pltpu.CompilerParams(
            use_tc_tiling_on_sc=False, needs_layout_passes=False),
    )
    return run(sid3, pid3, sw_lin, pw_lin)


def kernel(sid, pid, scientist_weight, paper_weight):
    sid3 = sid.astype(jnp.int32).reshape(_NW, _NCHUNK, _CHUNK)
    pid3 = pid.astype(jnp.int32).reshape(_NW, _NCHUNK, _CHUNK)

    sw_lin = _sc_transpose_one(scientist_weight.T).reshape(_N, _D)
    pw_lin = _sc_transpose_one(paper_weight.T).reshape(_N, _D)

    out = _sc_gather_dot(sid3, pid3, sw_lin, pw_lin)
    return out.reshape(_BATCH)
